# baseline (speedup 1.0000x reference)
import functools

import jax
import jax.numpy as jnp
from jax import lax
from jax.experimental import pallas as pl
from jax.experimental.pallas import tpu as pltpu
from jax.experimental.pallas import tpu_sc as plsc

N_ATOMS = 100000
NUM_TYPES = 64
LANES = 16
NUM_WORKERS = 32
CHUNK = 3200
LAST_BASE = N_ATOMS - CHUNK

_mesh = plsc.VectorSubcoreMesh(core_axis_name="c", subcore_axis_name="s")


@functools.partial(
    pl.kernel,
    mesh=_mesh,
    out_type=(
        jax.ShapeDtypeStruct((N_ATOMS,), jnp.float32),
        jax.ShapeDtypeStruct((N_ATOMS,), jnp.float32),
    ),
    compiler_params=pltpu.CompilerParams(needs_layout_passes=False),
    scratch_types=[
        pltpu.VMEM((CHUNK,), jnp.int32),
        pltpu.VMEM((CHUNK,), jnp.float32),
        pltpu.VMEM((CHUNK,), jnp.float32),
        pltpu.VMEM((NUM_TYPES,), jnp.float32),
        pltpu.VMEM((NUM_TYPES,), jnp.float32),
        pltpu.SemaphoreType.DMA,
    ],
)
def _gather_tables_sc(t_hbm, scales_hbm, shifts_hbm, s_out_hbm, b_out_hbm,
                      idx_v, s_v, b_v, sc_v, sh_v, sem):
    wid = lax.axis_index("s") * 2 + lax.axis_index("c")
    base = jnp.minimum(wid * CHUNK, LAST_BASE)

    c1 = pltpu.async_copy(t_hbm.at[pl.ds(base, CHUNK)], idx_v, sem)
    c2 = pltpu.async_copy(scales_hbm, sc_v, sem)
    c3 = pltpu.async_copy(shifts_hbm, sh_v, sem)
    c1.wait()
    c2.wait()
    c3.wait()

    @plsc.parallel_loop(0, CHUNK, LANES, unroll=8)
    def _(i):
        sl = pl.ds(i, LANES)
        idx = idx_v[sl]
        s_v[sl] = plsc.load_gather(sc_v, [idx])
        b_v[sl] = plsc.load_gather(sh_v, [idx])

    c4 = pltpu.async_copy(s_v, s_out_hbm.at[pl.ds(base, CHUNK)], sem)
    c5 = pltpu.async_copy(b_v, b_out_hbm.at[pl.ds(base, CHUNK)], sem)
    c4.wait()
    c5.wait()


def kernel(atomic_energy, atom_types, scales, shifts):
    t = atom_types.reshape(-1).astype(jnp.int32)
    s, b = _gather_tables_sc(t, scales.astype(jnp.float32),
                             shifts.astype(jnp.float32))
    x = atomic_energy.astype(jnp.float32)
    return b[:, None] + s[:, None] * x
